# baseline (speedup 1.0000x reference)
import functools
import math

import jax
import jax.numpy as jnp
import numpy as np
from jax import lax
from jax.experimental import pallas as pl
from jax.experimental.pallas import tpu as pltpu
from jax.experimental.pallas import tpu_sc as plsc

VOCAB = 32000
D = 2048
SEQ = 2048
BATCH = 4
L = 16
NC = 2
NS = 16
NW = NC * NS
POS_PER_W = SEQ // NW
CHUNK = 16
NCHUNK = POS_PER_W // CHUNK
SCALE = math.sqrt(float(D))
NVJ = D // L


def _make_pe_np():
    position = np.arange(0, SEQ).astype(np.float32)[:, None]
    div_term = np.exp(
        np.arange(0, D, 2).astype(np.float32) * (-math.log(10000.0) / D)
    )
    pe = np.zeros((SEQ, D), dtype=np.float32)
    pe[:, 0::2] = np.sin(position * div_term)
    pe[:, 1::2] = np.cos(position * div_term)
    return pe


def _rsqrt16(v):
    i = lax.bitcast_convert_type(v, jnp.int32)
    i = jnp.int32(0x5F3759DF) - lax.shift_right_logical(i, 1)
    y = lax.bitcast_convert_type(i, jnp.float32)
    for _ in range(3):
        y = y * (1.5 - 0.5 * v * y * y)
    return y


_mesh = plsc.VectorSubcoreMesh(core_axis_name="c", subcore_axis_name="s")


@functools.partial(
    pl.kernel,
    out_type=jax.ShapeDtypeStruct((BATCH, SEQ, D), jnp.float32),
    mesh=_mesh,
    compiler_params=pltpu.CompilerParams(needs_layout_passes=False),
    scratch_types=[
        pltpu.VMEM((CHUNK,), jnp.int32),
        pltpu.VMEM((CHUNK,), jnp.int32),
        pltpu.VMEM((CHUNK, D), jnp.float32),
        pltpu.VMEM((CHUNK, D), jnp.float32),
        pltpu.VMEM((D,), jnp.float32),
        pltpu.VMEM((D,), jnp.float32),
        pltpu.SemaphoreType.DMA,
    ],
)
def _sc_encoder(ids, mask, table, gamma, beta, pe, out,
                idx_v, mask_v, rows_v, pe_v, g_v, b_v, sem):
    wid = lax.axis_index("s") * NC + lax.axis_index("c")
    s0 = wid * POS_PER_W

    pltpu.sync_copy(gamma, g_v)
    pltpu.sync_copy(beta, b_v)
    lanes = lax.iota(jnp.int32, L)

    def token_body(t, _):
        mf = mask_v[...].astype(jnp.float32)
        m = jnp.sum(jnp.where(lanes == t, mf, 0.0))
        m_splat = jnp.full((L,), m, jnp.float32)

        def p1(j, carry):
            s1, s2 = carry
            r = rows_v[t, pl.ds(j * L, L)]
            p = pe_v[t, pl.ds(j * L, L)]
            x = (r * SCALE + p) * m_splat
            rows_v[t, pl.ds(j * L, L)] = x
            return (s1 + x, s2 + x * x)

        z = jnp.zeros((L,), jnp.float32)
        s1, s2 = lax.fori_loop(0, NVJ, p1, (z, z))
        mean = jnp.sum(s1) * (1.0 / D)
        ex2 = jnp.sum(s2) * (1.0 / D)
        var = ex2 - mean * mean
        inv = _rsqrt16(jnp.full((L,), var + 1e-5, jnp.float32))
        mean_splat = jnp.full((L,), mean, jnp.float32)

        def p2(j, carry):
            x = rows_v[t, pl.ds(j * L, L)]
            g = g_v[pl.ds(j * L, L)]
            bb = b_v[pl.ds(j * L, L)]
            rows_v[t, pl.ds(j * L, L)] = (x - mean_splat) * inv * g + bb
            return carry

        lax.fori_loop(0, NVJ, p2, 0)
        return _

    def c_body(c, _):
        off = s0 + c * CHUNK
        pltpu.sync_copy(pe.at[pl.ds(off, CHUNK), :], pe_v)

        def b_body(b, carry):
            pltpu.sync_copy(ids.at[b, pl.ds(off, CHUNK)], idx_v)
            pltpu.sync_copy(mask.at[b, pl.ds(off, CHUNK)], mask_v)
            pltpu.async_copy(table.at[idx_v], rows_v, sem).wait()
            lax.fori_loop(0, CHUNK, token_body, 0)
            pltpu.sync_copy(rows_v, out.at[b, pl.ds(off, CHUNK), :])
            return carry

        lax.fori_loop(0, BATCH, b_body, 0)
        return _

    lax.fori_loop(0, NCHUNK, c_body, 0)


def kernel(input_ids, attention_mask, table, gamma, beta):
    pe = jnp.asarray(_make_pe_np())
    return _sc_encoder(input_ids, attention_mask, table, gamma, beta, pe)
